# baseline (speedup 1.0000x reference)
import functools

import jax
import jax.numpy as jnp
from jax import lax
from jax.experimental import pallas as pl
from jax.experimental.pallas import tpu as pltpu
from jax.experimental.pallas import tpu_sc as plsc


def _make_sc_spmm(N, D, NC, NS, K, C, KH):
  NH = K // KH
  HW = D // 2
  HB = D // 32
  NP = N // 2
  PZ = 40

  mesh = plsc.VectorSubcoreMesh(core_axis_name="c", subcore_axis_name="s")

  @functools.partial(
      pl.kernel,
      out_type=jax.ShapeDtypeStruct((NC, NP, D), jnp.float32),
      mesh=mesh,
      scratch_types=[
          pltpu.VMEM((KH, C), jnp.int32),
          pltpu.VMEM((KH, C), jnp.int32),
          pltpu.VMEM((KH, C), jnp.float32),
          pltpu.VMEM((C, D), jnp.float32),
          pltpu.VMEM((C, D), jnp.float32),
          pltpu.VMEM((C, D), jnp.float32),
          pltpu.VMEM_SHARED((NP, D), jnp.float32),
          pltpu.VMEM_SHARED((NP, D), jnp.float32),
          pltpu.SemaphoreType.DMA,
          pltpu.SemaphoreType.DMA,
          pltpu.SemaphoreType.DMA,
          pltpu.SemaphoreType.DMA,
          pltpu.SemaphoreType.DMA,
          pltpu.SemaphoreType.DMA,
      ],
      compiler_params=pltpu.CompilerParams(needs_layout_passes=False),
  )
  def sc_spmm(xp_hbm, row_hbm, col_hbm, val_hbm, out_hbm,
              cidx, ridx, vals, b0, b1, b2, xsp, acc,
              g0, g1, g2, s0, s1, s2):
    cid = lax.axis_index("c")
    sid = lax.axis_index("s")

    def zrow(i, _):
      for j in range(D // 16):
        b0[i, pl.ds(j * 16, 16)] = jnp.zeros((16,), jnp.float32)
      return 0
    lax.fori_loop(0, C, zrow, 0)
    nz_full = NP // C
    for m in range((nz_full + 1 + NS - 1) // NS):
      idx = sid + NS * m
      @pl.when(idx < nz_full)
      def _():
        pltpu.sync_copy(b0, acc.at[pl.ds(pl.multiple_of(idx * C, 8), C)])
      if NP % C:
        @pl.when(idx == nz_full)
        def _():
          pltpu.sync_copy(b0.at[pl.ds(0, NP % C)],
                          acc.at[pl.ds(pl.multiple_of(nz_full * C, 8),
                                       NP % C)])
    for m in range((NP // PZ + NS - 1) // NS):
      idx = sid + NS * m
      @pl.when(idx < NP // PZ)
      def _():
        off = pl.multiple_of(idx * PZ, 8)
        pltpu.sync_copy(xp_hbm.at[cid, pl.ds(off, PZ)],
                        xsp.at[pl.ds(off, PZ)])
    plsc.subcore_barrier()

    lane = lax.iota(jnp.int32, 16)
    zero16 = jnp.zeros((16,), jnp.float32)

    def scale(b, k):
      kvec = jnp.zeros((16,), jnp.int32) + k
      def srow(i, _):
        ivec = jnp.zeros((16,), jnp.int32) + i
        sv = plsc.load_gather(vals, [kvec, ivec])
        av = jnp.abs(sv)
        pvec = jnp.where(sv < 0.0, HW, 0) + lane
        qbase = jnp.where(av >= 2.0, HW, 0)
        v = jnp.where(av >= 2.0, av - 2.0, av)
        qvec = qbase + lane
        zvec = (qbase ^ HW) + lane
        for j in range(HB):
          w = plsc.load_gather(b, [ivec, pvec + j * 16])
          plsc.store_scatter(b, [ivec, qvec + j * 16], w * v)
          plsc.store_scatter(b, [ivec, zvec + j * 16], zero16)
        return 0
      lax.fori_loop(0, C, srow, 0)

    def gather(k, b, sem):
      pltpu.async_copy(xsp.at[cidx.at[k]], b, sem)

    def gwait(k, b, sem):
      pltpu.make_async_copy(xsp.at[cidx.at[k]], b, sem).wait()

    def scat(k, b, sem):
      pltpu.async_copy(b, acc.at[ridx.at[k]], sem, add=True)

    def swait(k, b, sem):
      pltpu.make_async_copy(b, acc.at[ridx.at[k]], sem).wait()

    def body(k3, _):
      k = 3 * k3
      bufs = ((b0, g0, s0), (b1, g1, s1), (b2, g2, s2))
      for t in range(3):
        b, g, s = bufs[t]
        gwait(k + t, b, g)
        scale(b, k + t)
        scat(k + t, b, s)
        swait(k + t, b, s)
        @pl.when(k + t + 3 < KH)
        def _():
          gather(k + t + 3, b, g)
      return 0

    for h in range(NH):
      slab = pl.multiple_of(sid * K + h * KH, 8)
      pltpu.sync_copy(col_hbm.at[pl.ds(slab, KH)], cidx)
      pltpu.sync_copy(row_hbm.at[pl.ds(slab, KH)], ridx)
      pltpu.sync_copy(val_hbm.at[pl.ds(slab, KH)], vals)
      gather(0, b0, g0)
      gather(1, b1, g1)
      gather(2, b2, g2)
      lax.fori_loop(0, KH // 3, body, 0)

    plsc.subcore_barrier()

    for m in range((NP // PZ + NS - 1) // NS):
      idx = sid + NS * m
      @pl.when(idx < NP // PZ)
      def _():
        off = pl.multiple_of(idx * PZ, 8)
        pltpu.sync_copy(acc.at[pl.ds(off, PZ)],
                        out_hbm.at[cid, pl.ds(off, PZ)])

  return sc_spmm


def _tc_matmul_prelu(ph, W, prelu_a, N, D, NC):
  BR = 1000
  grid = (N // BR,)
  HD = D // 2

  def body(a_ref, p0_ref, p1_ref, w_ref, o_ref):
    s = jnp.concatenate([p0_ref[0], p1_ref[0]], axis=-1)
    h = jnp.dot(s, w_ref[...], preferred_element_type=jnp.float32)
    a = a_ref[0, 0]
    o_ref[...] = jnp.where(h >= 0, h, a * h)

  return pl.pallas_call(
      body,
      grid=grid,
      in_specs=[
          pl.BlockSpec((1, 1), lambda i: (0, 0)),
          pl.BlockSpec((1, BR, HD), lambda i: (0, i, 0)),
          pl.BlockSpec((1, BR, HD), lambda i: (1, i, 0)),
          pl.BlockSpec((D, D), lambda i: (0, 0)),
      ],
      out_specs=pl.BlockSpec((BR, D), lambda i: (i, 0)),
      out_shape=jax.ShapeDtypeStruct((N, D), jnp.float32),
  )(prelu_a.reshape(1, 1), ph, ph, W)


def kernel(x, edge_index, adj_vals, W, prelu_a):
  N, D = x.shape
  E = adj_vals.shape[0]
  info = plsc.get_sparse_core_info()
  NC, NS = info.num_cores, info.num_subcores

  C = 96
  KH = 24
  K = -(-E // (NS * C))
  K = -(-K // KH) * KH
  EP = NS * K * C
  pad = EP - E

  xp = jnp.stack([x[:, : D // 2].reshape(N // 2, D),
                  x[:, D // 2:].reshape(N // 2, D)])

  row = edge_index[0].astype(jnp.int32)
  col = edge_index[1].astype(jnp.int32)
  if pad:
    zpad_i = jnp.zeros((pad,), jnp.int32)
    row = jnp.concatenate([row, zpad_i])
    col = jnp.concatenate([col, zpad_i])
    adj_vals = jnp.concatenate([adj_vals, jnp.zeros((pad,), jnp.float32)])
  sval = adj_vals + jnp.where((row & 1) == 1, 2.0, 0.0)
  sval = jnp.where((col & 1) == 1, -sval, sval)
  chalf = (col >> 1).reshape(NS * K, C)
  rhalf = (row >> 1).reshape(NS * K, C)
  sval = sval.reshape(NS * K, C)

  sc_spmm = _make_sc_spmm(N, D, NC, NS, K, C, KH)
  ph = sc_spmm(xp, rhalf, chalf, sval)
  return _tc_matmul_prelu(ph.reshape(NC, N, D // 2), W, prelu_a, N, D, NC)


# speedup vs baseline: 2.2107x; 2.2107x over previous
import functools

import jax
import jax.numpy as jnp
from jax import lax
from jax.experimental import pallas as pl
from jax.experimental.pallas import tpu as pltpu
from jax.experimental.pallas import tpu_sc as plsc


def _make_sc_spmm(N, D, NC, NS, K, C, KH):
  NH = K // KH
  HW = D // 2
  HB = D // 32
  NP = N // 2
  PZ = 40

  mesh = plsc.VectorSubcoreMesh(core_axis_name="c", subcore_axis_name="s")

  @functools.partial(
      pl.kernel,
      out_type=jax.ShapeDtypeStruct((NC, NP, D), jnp.float32),
      mesh=mesh,
      scratch_types=[
          pltpu.VMEM((KH, C), jnp.int32),
          pltpu.VMEM((KH, C), jnp.int32),
          pltpu.VMEM((KH, C), jnp.float32),
          pltpu.VMEM((C, D), jnp.float32),
          pltpu.VMEM((C, D), jnp.float32),
          pltpu.VMEM((C, D), jnp.float32),
          pltpu.VMEM_SHARED((NP, D), jnp.float32),
          pltpu.VMEM_SHARED((NP, D), jnp.float32),
          pltpu.SemaphoreType.DMA,
          pltpu.SemaphoreType.DMA,
          pltpu.SemaphoreType.DMA,
          pltpu.SemaphoreType.DMA,
          pltpu.SemaphoreType.DMA,
          pltpu.SemaphoreType.DMA,
      ],
      compiler_params=pltpu.CompilerParams(needs_layout_passes=False),
  )
  def sc_spmm(xp_hbm, row_hbm, col_hbm, val_hbm, out_hbm,
              cidx, ridx, vals, b0, b1, b2, xsp, acc,
              g0, g1, g2, s0, s1, s2):
    cid = lax.axis_index("c")
    sid = lax.axis_index("s")

    def zrow(i, _):
      for j in range(D // 16):
        b0[i, pl.ds(j * 16, 16)] = jnp.zeros((16,), jnp.float32)
      return 0
    lax.fori_loop(0, C, zrow, 0)
    nz_full = NP // C
    for m in range((nz_full + 1 + NS - 1) // NS):
      idx = sid + NS * m
      @pl.when(idx < nz_full)
      def _():
        pltpu.sync_copy(b0, acc.at[pl.ds(pl.multiple_of(idx * C, 8), C)])
      if NP % C:
        @pl.when(idx == nz_full)
        def _():
          pltpu.sync_copy(b0.at[pl.ds(0, NP % C)],
                          acc.at[pl.ds(pl.multiple_of(nz_full * C, 8),
                                       NP % C)])
    for m in range((NP // PZ + NS - 1) // NS):
      idx = sid + NS * m
      @pl.when(idx < NP // PZ)
      def _():
        off = pl.multiple_of(idx * PZ, 8)
        pltpu.sync_copy(xp_hbm.at[cid, pl.ds(off, PZ)],
                        xsp.at[pl.ds(off, PZ)])
    plsc.subcore_barrier()

    lane = lax.iota(jnp.int32, 16)
    zero16 = jnp.zeros((16,), jnp.float32)

    def scale(b, k):
      kvec = jnp.zeros((16,), jnp.int32) + k
      @functools.partial(plsc.parallel_loop, 0, C, unroll=8)
      def srow(i):
        ivec = jnp.zeros((16,), jnp.int32) + i
        sv = plsc.load_gather(vals, [kvec, ivec])
        av = jnp.abs(sv)
        pvec = jnp.where(sv < 0.0, HW, 0) + lane
        qbase = jnp.where(av >= 2.0, HW, 0)
        v = jnp.where(av >= 2.0, av - 2.0, av)
        qvec = qbase + lane
        zvec = (qbase ^ HW) + lane
        for j in range(HB):
          w = plsc.load_gather(b, [ivec, pvec + j * 16])
          plsc.store_scatter(b, [ivec, qvec + j * 16], w * v)
          plsc.store_scatter(b, [ivec, zvec + j * 16], zero16)

    def gather(k, b, sem):
      pltpu.async_copy(xsp.at[cidx.at[k]], b, sem)

    def gwait(k, b, sem):
      pltpu.make_async_copy(xsp.at[cidx.at[k]], b, sem).wait()

    def scat(k, b, sem):
      pltpu.async_copy(b, acc.at[ridx.at[k]], sem, add=True)

    def swait(k, b, sem):
      pltpu.make_async_copy(b, acc.at[ridx.at[k]], sem).wait()

    def body(k3, _):
      k = 3 * k3
      bufs = ((b0, g0, s0), (b1, g1, s1), (b2, g2, s2))
      for t in range(3):
        b, g, s = bufs[t]
        gwait(k + t, b, g)
        scale(b, k + t)
        scat(k + t, b, s)
        swait(k + t, b, s)
        @pl.when(k + t + 3 < KH)
        def _():
          gather(k + t + 3, b, g)
      return 0

    for h in range(NH):
      slab = pl.multiple_of(sid * K + h * KH, 8)
      pltpu.sync_copy(col_hbm.at[pl.ds(slab, KH)], cidx)
      pltpu.sync_copy(row_hbm.at[pl.ds(slab, KH)], ridx)
      pltpu.sync_copy(val_hbm.at[pl.ds(slab, KH)], vals)
      gather(0, b0, g0)
      gather(1, b1, g1)
      gather(2, b2, g2)
      lax.fori_loop(0, KH // 3, body, 0)

    plsc.subcore_barrier()

    for m in range((NP // PZ + NS - 1) // NS):
      idx = sid + NS * m
      @pl.when(idx < NP // PZ)
      def _():
        off = pl.multiple_of(idx * PZ, 8)
        pltpu.sync_copy(acc.at[pl.ds(off, PZ)],
                        out_hbm.at[cid, pl.ds(off, PZ)])

  return sc_spmm


def _tc_matmul_prelu(ph, W, prelu_a, N, D, NC):
  BR = 1000
  grid = (N // BR,)
  HD = D // 2

  def body(a_ref, p0_ref, p1_ref, w_ref, o_ref):
    s = jnp.concatenate([p0_ref[0], p1_ref[0]], axis=-1)
    h = jnp.dot(s, w_ref[...], preferred_element_type=jnp.float32)
    a = a_ref[0, 0]
    o_ref[...] = jnp.where(h >= 0, h, a * h)

  return pl.pallas_call(
      body,
      grid=grid,
      in_specs=[
          pl.BlockSpec((1, 1), lambda i: (0, 0)),
          pl.BlockSpec((1, BR, HD), lambda i: (0, i, 0)),
          pl.BlockSpec((1, BR, HD), lambda i: (1, i, 0)),
          pl.BlockSpec((D, D), lambda i: (0, 0)),
      ],
      out_specs=pl.BlockSpec((BR, D), lambda i: (i, 0)),
      out_shape=jax.ShapeDtypeStruct((N, D), jnp.float32),
  )(prelu_a.reshape(1, 1), ph, ph, W)


def kernel(x, edge_index, adj_vals, W, prelu_a):
  N, D = x.shape
  E = adj_vals.shape[0]
  info = plsc.get_sparse_core_info()
  NC, NS = info.num_cores, info.num_subcores

  C = 96
  KH = 24
  K = -(-E // (NS * C))
  K = -(-K // KH) * KH
  EP = NS * K * C
  pad = EP - E

  xp = jnp.stack([x[:, : D // 2].reshape(N // 2, D),
                  x[:, D // 2:].reshape(N // 2, D)])

  row = edge_index[0].astype(jnp.int32)
  col = edge_index[1].astype(jnp.int32)
  if pad:
    zpad_i = jnp.zeros((pad,), jnp.int32)
    row = jnp.concatenate([row, zpad_i])
    col = jnp.concatenate([col, zpad_i])
    adj_vals = jnp.concatenate([adj_vals, jnp.zeros((pad,), jnp.float32)])
  sval = adj_vals + jnp.where((row & 1) == 1, 2.0, 0.0)
  sval = jnp.where((col & 1) == 1, -sval, sval)
  chalf = (col >> 1).reshape(NS * K, C)
  rhalf = (row >> 1).reshape(NS * K, C)
  sval = sval.reshape(NS * K, C)

  sc_spmm = _make_sc_spmm(N, D, NC, NS, K, C, KH)
  ph = sc_spmm(xp, rhalf, chalf, sval)
  return _tc_matmul_prelu(ph.reshape(NC, N, D // 2), W, prelu_a, N, D, NC)


# speedup vs baseline: 2.2226x; 1.0054x over previous
import functools

import jax
import jax.numpy as jnp
from jax import lax
from jax.experimental import pallas as pl
from jax.experimental.pallas import tpu as pltpu
from jax.experimental.pallas import tpu_sc as plsc


def _make_sc_spmm(N, D, NC, NS, K, C, KH):
  NH = K // KH
  HW = D // 2
  HB = D // 32
  NP = N // 2
  PZ = 40

  mesh = plsc.VectorSubcoreMesh(core_axis_name="c", subcore_axis_name="s")

  @functools.partial(
      pl.kernel,
      out_type=jax.ShapeDtypeStruct((NC, NP, D), jnp.float32),
      mesh=mesh,
      scratch_types=[
          pltpu.VMEM((KH, C), jnp.int32),
          pltpu.VMEM((KH, C), jnp.int32),
          pltpu.VMEM((KH, C), jnp.float32),
          pltpu.VMEM((C, D), jnp.float32),
          pltpu.VMEM((C, D), jnp.float32),
          pltpu.VMEM((C, D), jnp.float32),
          pltpu.VMEM_SHARED((NP, D), jnp.float32),
          pltpu.VMEM_SHARED((NP, D), jnp.float32),
          pltpu.SemaphoreType.DMA,
          pltpu.SemaphoreType.DMA,
          pltpu.SemaphoreType.DMA,
          pltpu.SemaphoreType.DMA,
          pltpu.SemaphoreType.DMA,
          pltpu.SemaphoreType.DMA,
      ],
      compiler_params=pltpu.CompilerParams(needs_layout_passes=False),
  )
  def sc_spmm(xp_hbm, row_hbm, col_hbm, val_hbm, out_hbm,
              cidx, ridx, vals, b0, b1, b2, xsp, acc,
              g0, g1, g2, s0, s1, s2):
    cid = lax.axis_index("c")
    sid = lax.axis_index("s")

    def zrow(i, _):
      for j in range(D // 16):
        b0[i, pl.ds(j * 16, 16)] = jnp.zeros((16,), jnp.float32)
      return 0
    lax.fori_loop(0, C, zrow, 0)
    nz_full = NP // C
    for m in range((nz_full + 1 + NS - 1) // NS):
      idx = sid + NS * m
      @pl.when(idx < nz_full)
      def _():
        pltpu.sync_copy(b0, acc.at[pl.ds(pl.multiple_of(idx * C, 8), C)])
      if NP % C:
        @pl.when(idx == nz_full)
        def _():
          pltpu.sync_copy(b0.at[pl.ds(0, NP % C)],
                          acc.at[pl.ds(pl.multiple_of(nz_full * C, 8),
                                       NP % C)])
    for m in range((NP // PZ + NS - 1) // NS):
      idx = sid + NS * m
      @pl.when(idx < NP // PZ)
      def _():
        off = pl.multiple_of(idx * PZ, 8)
        pltpu.sync_copy(xp_hbm.at[cid, pl.ds(off, PZ)],
                        xsp.at[pl.ds(off, PZ)])
    plsc.subcore_barrier()

    lane = lax.iota(jnp.int32, 16)
    zero16 = jnp.zeros((16,), jnp.float32)

    def scale(b, k):
      kvec = jnp.zeros((16,), jnp.int32) + k
      @functools.partial(plsc.parallel_loop, 0, C, unroll=8)
      def srow(i):
        ivec = jnp.zeros((16,), jnp.int32) + i
        sv = plsc.load_gather(vals, [kvec, ivec])
        av = jnp.abs(sv)
        pvec = jnp.where(sv < 0.0, HW, 0) + lane
        qbase = jnp.where(av >= 2.0, HW, 0)
        v = jnp.where(av >= 2.0, av - 2.0, av)
        qvec = qbase + lane
        zvec = (qbase ^ HW) + lane
        for j in range(HB):
          w = plsc.load_gather(b, [ivec, pvec + j * 16])
          plsc.store_scatter(b, [ivec, qvec + j * 16], w * v)
          plsc.store_scatter(b, [ivec, zvec + j * 16], w * 0.0)

    def gather(k, b, sem):
      pltpu.async_copy(xsp.at[cidx.at[k]], b, sem)

    def gwait(k, b, sem):
      pltpu.make_async_copy(xsp.at[cidx.at[k]], b, sem).wait()

    def scat(k, b, sem):
      pltpu.async_copy(b, acc.at[ridx.at[k]], sem, add=True)

    def swait(k, b, sem):
      pltpu.make_async_copy(b, acc.at[ridx.at[k]], sem).wait()

    def body(k3, _):
      k = 3 * k3
      bufs = ((b0, g0, s0), (b1, g1, s1), (b2, g2, s2))
      for t in range(3):
        b, g, s = bufs[t]
        gwait(k + t, b, g)
        scale(b, k + t)
        scat(k + t, b, s)
        swait(k + t, b, s)
        @pl.when(k + t + 3 < KH)
        def _():
          gather(k + t + 3, b, g)
      return 0

    for h in range(NH):
      slab = pl.multiple_of(sid * K + h * KH, 8)
      pltpu.sync_copy(col_hbm.at[pl.ds(slab, KH)], cidx)
      pltpu.sync_copy(row_hbm.at[pl.ds(slab, KH)], ridx)
      pltpu.sync_copy(val_hbm.at[pl.ds(slab, KH)], vals)
      gather(0, b0, g0)
      gather(1, b1, g1)
      gather(2, b2, g2)
      lax.fori_loop(0, KH // 3, body, 0)

    plsc.subcore_barrier()

    for m in range((NP // PZ + NS - 1) // NS):
      idx = sid + NS * m
      @pl.when(idx < NP // PZ)
      def _():
        off = pl.multiple_of(idx * PZ, 8)
        pltpu.sync_copy(acc.at[pl.ds(off, PZ)],
                        out_hbm.at[cid, pl.ds(off, PZ)])

  return sc_spmm


def _tc_matmul_prelu(ph, W, prelu_a, N, D, NC):
  BR = 1000
  grid = (N // BR,)
  HD = D // 2

  def body(a_ref, p0_ref, p1_ref, w_ref, o_ref):
    s = jnp.concatenate([p0_ref[0], p1_ref[0]], axis=-1)
    h = jnp.dot(s, w_ref[...], preferred_element_type=jnp.float32)
    a = a_ref[0, 0]
    o_ref[...] = jnp.where(h >= 0, h, a * h)

  return pl.pallas_call(
      body,
      grid=grid,
      in_specs=[
          pl.BlockSpec((1, 1), lambda i: (0, 0)),
          pl.BlockSpec((1, BR, HD), lambda i: (0, i, 0)),
          pl.BlockSpec((1, BR, HD), lambda i: (1, i, 0)),
          pl.BlockSpec((D, D), lambda i: (0, 0)),
      ],
      out_specs=pl.BlockSpec((BR, D), lambda i: (i, 0)),
      out_shape=jax.ShapeDtypeStruct((N, D), jnp.float32),
  )(prelu_a.reshape(1, 1), ph, ph, W)


def kernel(x, edge_index, adj_vals, W, prelu_a):
  N, D = x.shape
  E = adj_vals.shape[0]
  info = plsc.get_sparse_core_info()
  NC, NS = info.num_cores, info.num_subcores

  C = 96
  KH = 24
  K = -(-E // (NS * C))
  K = -(-K // KH) * KH
  EP = NS * K * C
  pad = EP - E

  xp = jnp.stack([x[:, : D // 2].reshape(N // 2, D),
                  x[:, D // 2:].reshape(N // 2, D)])

  row = edge_index[0].astype(jnp.int32)
  col = edge_index[1].astype(jnp.int32)
  if pad:
    zpad_i = jnp.zeros((pad,), jnp.int32)
    row = jnp.concatenate([row, zpad_i])
    col = jnp.concatenate([col, zpad_i])
    adj_vals = jnp.concatenate([adj_vals, jnp.zeros((pad,), jnp.float32)])
  sval = adj_vals + jnp.where((row & 1) == 1, 2.0, 0.0)
  sval = jnp.where((col & 1) == 1, -sval, sval)
  chalf = (col >> 1).reshape(NS * K, C)
  rhalf = (row >> 1).reshape(NS * K, C)
  sval = sval.reshape(NS * K, C)

  sc_spmm = _make_sc_spmm(N, D, NC, NS, K, C, KH)
  ph = sc_spmm(xp, rhalf, chalf, sval)
  return _tc_matmul_prelu(ph.reshape(NC, N, D // 2), W, prelu_a, N, D, NC)


# speedup vs baseline: 2.6290x; 1.1828x over previous
import functools

import jax
import jax.numpy as jnp
from jax import lax
from jax.experimental import pallas as pl
from jax.experimental.pallas import tpu as pltpu
from jax.experimental.pallas import tpu_sc as plsc


def _make_sc_spmm(N, D, NC, NS, K, C, KH):
  NH = K // KH
  HW = D // 2
  HB = D // 32
  NP = N // 2
  PZ = 40

  mesh = plsc.VectorSubcoreMesh(core_axis_name="c", subcore_axis_name="s")

  @functools.partial(
      pl.kernel,
      out_type=jax.ShapeDtypeStruct((NC, NP, D), jnp.float32),
      mesh=mesh,
      scratch_types=[
          pltpu.VMEM((KH, C), jnp.int32),
          pltpu.VMEM((KH, C), jnp.int32),
          pltpu.VMEM((KH, C), jnp.float32),
          pltpu.VMEM((C, D), jnp.float32),
          pltpu.VMEM((C, D), jnp.float32),
          pltpu.VMEM((C, D), jnp.float32),
          pltpu.VMEM((C, D), jnp.float32),
          pltpu.VMEM_SHARED((NP, D), jnp.float32),
          pltpu.VMEM_SHARED((NP, D), jnp.float32),
          pltpu.SemaphoreType.DMA,
          pltpu.SemaphoreType.DMA,
          pltpu.SemaphoreType.DMA,
          pltpu.SemaphoreType.DMA,
      ],
      compiler_params=pltpu.CompilerParams(needs_layout_passes=False),
  )
  def sc_spmm(xp_hbm, row_hbm, col_hbm, val_hbm, out_hbm,
              cidx, ridx, vals, b0, b1, t0, t1, xsp, acc,
              g0, g1, s0, s1):
    cid = lax.axis_index("c")
    sid = lax.axis_index("s")

    def zrow(i, _):
      for j in range(D // 16):
        b0[i, pl.ds(j * 16, 16)] = jnp.zeros((16,), jnp.float32)
      return 0
    lax.fori_loop(0, C, zrow, 0)
    nz_full = NP // C
    for m in range((nz_full + 1 + NS - 1) // NS):
      idx = sid + NS * m
      @pl.when(idx < nz_full)
      def _():
        pltpu.sync_copy(b0, acc.at[pl.ds(pl.multiple_of(idx * C, 8), C)])
      if NP % C:
        @pl.when(idx == nz_full)
        def _():
          pltpu.sync_copy(b0.at[pl.ds(0, NP % C)],
                          acc.at[pl.ds(pl.multiple_of(nz_full * C, 8),
                                       NP % C)])
    for m in range((NP // PZ + NS - 1) // NS):
      idx = sid + NS * m
      @pl.when(idx < NP // PZ)
      def _():
        off = pl.multiple_of(idx * PZ, 8)
        pltpu.sync_copy(xp_hbm.at[cid, pl.ds(off, PZ)],
                        xsp.at[pl.ds(off, PZ)])
    plsc.subcore_barrier()

    lane = lax.iota(jnp.int32, 16)
    zero16 = jnp.zeros((16,), jnp.float32)

    def scale(src, dst, k):
      kvec = jnp.zeros((16,), jnp.int32) + k
      @functools.partial(plsc.parallel_loop, 0, C, unroll=8)
      def srow(i):
        ivec = jnp.zeros((16,), jnp.int32) + i
        sv = plsc.load_gather(vals, [kvec, ivec])
        av = jnp.abs(sv)
        pvec = jnp.where(sv < 0.0, HW, 0) + lane
        qbase = jnp.where(av >= 2.0, HW, 0)
        v = jnp.where(av >= 2.0, av - 2.0, av)
        qvec = qbase + lane
        zvec = (qbase ^ HW) + lane
        for j in range(HB):
          w = plsc.load_gather(src, [ivec, pvec + j * 16])
          plsc.store_scatter(dst, [ivec, qvec + j * 16], w * v)
          plsc.store_scatter(dst, [ivec, zvec + j * 16], zero16)

    def gather(k, b, sem):
      pltpu.async_copy(xsp.at[cidx.at[k]], b, sem)

    def gwait(k, b, sem):
      pltpu.make_async_copy(xsp.at[cidx.at[k]], b, sem).wait()

    def scat(k, t, sem):
      pltpu.async_copy(t, acc.at[ridx.at[k]], sem, add=True)

    def swait(k, t, sem):
      pltpu.make_async_copy(t, acc.at[ridx.at[k]], sem).wait()

    def body(k2, _):
      k = 2 * k2
      for u, (b, t, g, s) in enumerate(((b0, t0, g0, s0), (b1, t1, g1, s1))):
        ku = k + u
        gwait(ku, b, g)
        @pl.when(k2 > 0)
        def _():
          swait(ku - 2, t, s)
        scale(b, t, ku)
        scat(ku, t, s)
        @pl.when(ku + 2 < KH)
        def _():
          gather(ku + 2, b, g)
      return 0

    for h in range(NH):
      slab = pl.multiple_of(sid * K + h * KH, 8)
      pltpu.sync_copy(col_hbm.at[pl.ds(slab, KH)], cidx)
      pltpu.sync_copy(row_hbm.at[pl.ds(slab, KH)], ridx)
      pltpu.sync_copy(val_hbm.at[pl.ds(slab, KH)], vals)
      gather(0, b0, g0)
      gather(1, b1, g1)
      lax.fori_loop(0, KH // 2, body, 0)
      swait(KH - 2, t0, s0)
      swait(KH - 1, t1, s1)

    plsc.subcore_barrier()

    for m in range((NP // PZ + NS - 1) // NS):
      idx = sid + NS * m
      @pl.when(idx < NP // PZ)
      def _():
        off = pl.multiple_of(idx * PZ, 8)
        pltpu.sync_copy(acc.at[pl.ds(off, PZ)],
                        out_hbm.at[cid, pl.ds(off, PZ)])

  return sc_spmm


def _tc_matmul_prelu(ph, W, prelu_a, N, D, NC):
  BR = 1000
  grid = (N // BR,)
  HD = D // 2

  def body(a_ref, p0_ref, p1_ref, w_ref, o_ref):
    s = jnp.concatenate([p0_ref[0], p1_ref[0]], axis=-1)
    h = jnp.dot(s, w_ref[...], preferred_element_type=jnp.float32)
    a = a_ref[0, 0]
    o_ref[...] = jnp.where(h >= 0, h, a * h)

  return pl.pallas_call(
      body,
      grid=grid,
      in_specs=[
          pl.BlockSpec((1, 1), lambda i: (0, 0)),
          pl.BlockSpec((1, BR, HD), lambda i: (0, i, 0)),
          pl.BlockSpec((1, BR, HD), lambda i: (1, i, 0)),
          pl.BlockSpec((D, D), lambda i: (0, 0)),
      ],
      out_specs=pl.BlockSpec((BR, D), lambda i: (i, 0)),
      out_shape=jax.ShapeDtypeStruct((N, D), jnp.float32),
  )(prelu_a.reshape(1, 1), ph, ph, W)


def kernel(x, edge_index, adj_vals, W, prelu_a):
  N, D = x.shape
  E = adj_vals.shape[0]
  info = plsc.get_sparse_core_info()
  NC, NS = info.num_cores, info.num_subcores

  C = 80
  KH = 16
  K = -(-E // (NS * C))
  K = -(-K // KH) * KH
  EP = NS * K * C
  pad = EP - E

  xp = jnp.stack([x[:, : D // 2].reshape(N // 2, D),
                  x[:, D // 2:].reshape(N // 2, D)])

  row = edge_index[0].astype(jnp.int32)
  col = edge_index[1].astype(jnp.int32)
  if pad:
    zpad_i = jnp.zeros((pad,), jnp.int32)
    row = jnp.concatenate([row, zpad_i])
    col = jnp.concatenate([col, zpad_i])
    adj_vals = jnp.concatenate([adj_vals, jnp.zeros((pad,), jnp.float32)])
  sval = adj_vals + jnp.where((row & 1) == 1, 2.0, 0.0)
  sval = jnp.where((col & 1) == 1, -sval, sval)
  chalf = (col >> 1).reshape(NS * K, C)
  rhalf = (row >> 1).reshape(NS * K, C)
  sval = sval.reshape(NS * K, C)

  sc_spmm = _make_sc_spmm(N, D, NC, NS, K, C, KH)
  ph = sc_spmm(xp, rhalf, chalf, sval)
  return _tc_matmul_prelu(ph.reshape(NC, N, D // 2), W, prelu_a, N, D, NC)
